# baseline (speedup 1.0000x reference)
import jax
import jax.numpy as jnp
from jax import lax
from jax.experimental import pallas as pl
from jax.experimental.pallas import tpu as pltpu
from jax.experimental.pallas import tpu_sc as plsc

B = 16384
ED = 64
OUT_D = 3 * ED
SV, VV, TV = 3, 4, 3

NC, NS, L = 2, 16, 16
NW = NC * NS
BPW = B // NW
CHUNK = 128
NCHUNK = BPW // CHUNK
VPC = CHUNK // L
HALF = BPW // 2


def _body(sess_hbm, vol_hbm, trend_hbm, t0_hbm, t1_hbm, out_hbm,
          sidx_v, vidx_v, tidx_v, sv_v, t_v, bufa_v, bufb_v, sem):
    wid = lax.axis_index("s") * NC + lax.axis_index("c")
    base = wid * BPW

    pltpu.sync_copy(sess_hbm.at[pl.ds(base, BPW)], sidx_v)
    pltpu.sync_copy(vol_hbm.at[pl.ds(base, BPW)], vidx_v)
    pltpu.sync_copy(trend_hbm.at[pl.ds(base, BPW)], tidx_v)

    for i in range(BPW // L):
        s = sidx_v[pl.ds(i * L, L)]
        v = vidx_v[pl.ds(i * L, L)]
        t = tidx_v[pl.ds(i * L, L)]
        s = jnp.minimum(jnp.maximum(s, 0), SV - 1)
        v = jnp.minimum(jnp.maximum(v, 0), VV - 1)
        t = jnp.minimum(jnp.maximum(t, 0), TV - 1)
        sv_v[i // VPC, pl.ds((i % VPC) * L, L)] = s * VV + v
        t_v[i // VPC, pl.ds((i % VPC) * L, L)] = t

    for h in range(2):
        copies = []
        for j in range(2):
            c = 2 * h + j
            copies.append(pltpu.async_copy(
                t0_hbm.at[sv_v.at[c]], bufa_v.at[pl.ds(j * CHUNK, CHUNK)], sem))
            copies.append(pltpu.async_copy(
                t1_hbm.at[t_v.at[c]], bufb_v.at[pl.ds(j * CHUNK, CHUNK)], sem))
        for c in copies:
            c.wait()
        pltpu.sync_copy(
            bufa_v, out_hbm.at[pl.ds(base + h * HALF, HALF), pl.ds(0, 128)])
        pltpu.sync_copy(
            bufb_v, out_hbm.at[pl.ds(base + h * HALF, HALF), pl.ds(128, 128)])


def kernel(session_id, vol_regime_id, trend_regime_id,
           session_table, vol_table, trend_table):
    c = jnp.arange(SV * VV, dtype=jnp.int32)
    t0 = jnp.concatenate(
        [jnp.take(session_table, c // VV, axis=0),
         jnp.take(vol_table, c % VV, axis=0)],
        axis=-1,
    )
    t1 = jnp.pad(trend_table, ((0, 0), (0, 128 - ED)))

    run = pl.kernel(
        _body,
        mesh=plsc.VectorSubcoreMesh(core_axis_name="c", subcore_axis_name="s"),
        out_type=jax.ShapeDtypeStruct((B, 256), jnp.float32),
        scratch_types=[
            pltpu.VMEM((BPW,), jnp.int32),
            pltpu.VMEM((BPW,), jnp.int32),
            pltpu.VMEM((BPW,), jnp.int32),
            pltpu.VMEM((NCHUNK, CHUNK), jnp.int32),
            pltpu.VMEM((NCHUNK, CHUNK), jnp.int32),
            pltpu.VMEM((HALF, 128), jnp.float32),
            pltpu.VMEM((HALF, 128), jnp.float32),
            pltpu.SemaphoreType.DMA,
        ],
    )
    padded = run(
        session_id.astype(jnp.int32),
        vol_regime_id.astype(jnp.int32),
        trend_regime_id.astype(jnp.int32),
        t0,
        t1,
    )
    return padded[:, :OUT_D]


# speedup vs baseline: 4.4860x; 4.4860x over previous
import jax
import jax.numpy as jnp
from jax import lax
from jax.experimental import pallas as pl
from jax.experimental.pallas import tpu as pltpu
from jax.experimental.pallas import tpu_sc as plsc

B = 16384
ED = 64
OUT_D = 3 * ED
SV, VV, TV = 3, 4, 3
NCOMBO = SV * VV * TV
REP = 32

NC, NS, L = 2, 16, 16
NW = NC * NS
BPW = B // NW
CHUNK = 128
NCHUNK = BPW // CHUNK
VPC = CHUNK // L


def _body(sess_hbm, vol_hbm, trend_hbm, fused_hbm, out_hbm,
          sidx_v, vidx_v, tidx_v, combo_v, rows_v, sem):
    wid = lax.axis_index("s") * NC + lax.axis_index("c")
    base = wid * BPW

    pltpu.sync_copy(sess_hbm.at[pl.ds(base, BPW)], sidx_v)
    pltpu.sync_copy(vol_hbm.at[pl.ds(base, BPW)], vidx_v)
    pltpu.sync_copy(trend_hbm.at[pl.ds(base, BPW)], tidx_v)

    for i in range(BPW // L):
        s = sidx_v[pl.ds(i * L, L)]
        v = vidx_v[pl.ds(i * L, L)]
        t = tidx_v[pl.ds(i * L, L)]
        s = jnp.minimum(jnp.maximum(s, 0), SV - 1)
        v = jnp.minimum(jnp.maximum(v, 0), VV - 1)
        t = jnp.minimum(jnp.maximum(t, 0), TV - 1)
        combo = s * (VV * TV) + v * TV + t
        rep = (wid + i) % REP
        combo_v[i // VPC, pl.ds((i % VPC) * L, L)] = combo + NCOMBO * rep

    copies = [
        pltpu.async_copy(
            fused_hbm.at[combo_v.at[j]],
            rows_v.at[pl.ds(j * CHUNK, CHUNK)],
            sem,
        )
        for j in range(NCHUNK)
    ]
    for c in copies:
        c.wait()

    pltpu.sync_copy(rows_v, out_hbm.at[pl.ds(base, BPW)])


def kernel(session_id, vol_regime_id, trend_regime_id,
           session_table, vol_table, trend_table):
    c = jnp.arange(NCOMBO, dtype=jnp.int32)
    fused = jnp.concatenate(
        [
            jnp.take(session_table, c // (VV * TV), axis=0),
            jnp.take(vol_table, (c // TV) % VV, axis=0),
            jnp.take(trend_table, c % TV, axis=0),
        ],
        axis=-1,
    )
    fused = jnp.tile(fused, (REP, 1))

    run = pl.kernel(
        _body,
        mesh=plsc.VectorSubcoreMesh(core_axis_name="c", subcore_axis_name="s"),
        out_type=jax.ShapeDtypeStruct((B, OUT_D), jnp.float32),
        scratch_types=[
            pltpu.VMEM((BPW,), jnp.int32),
            pltpu.VMEM((BPW,), jnp.int32),
            pltpu.VMEM((BPW,), jnp.int32),
            pltpu.VMEM((NCHUNK, CHUNK), jnp.int32),
            pltpu.VMEM((BPW, OUT_D), jnp.float32),
            pltpu.SemaphoreType.DMA,
        ],
        compiler_params=pltpu.CompilerParams(use_tc_tiling_on_sc=False),
    )
    return run(
        session_id.astype(jnp.int32),
        vol_regime_id.astype(jnp.int32),
        trend_regime_id.astype(jnp.int32),
        fused,
    )
